# Initial kernel scaffold; baseline (speedup 1.0000x reference)
import jax
import jax.numpy as jnp
from jax.experimental import pallas as pl


def kernel(parent_node_embedding, children_index, batch_tree_mask, w_t, w_l, w_r, bias, ln_gamma, ln_beta):
    raise NotImplementedError("write your pallas kernel here")


# speedup vs baseline: 19.4496x; 19.4496x over previous
import functools
import jax
import jax.numpy as jnp
from jax.experimental import pallas as pl
from jax.experimental.pallas import tpu as pltpu

_B, _N, _C, _D = 256, 64, 32, 128
_GT = 8


def _tc_kernel(parent_ref, idx_ref, wt_ref, wl_ref, wr_ref, bias_ref,
               gamma_ref, beta_ref, out_ref):
    gt, n, d, c = _GT, _N, _D, _C
    rows = gt * n

    idx = idx_ref[...]
    mask = (idx != 0).astype(jnp.float32)
    ns = jnp.sum(mask, axis=1, keepdims=True)
    c_iota_i = jax.lax.broadcasted_iota(jnp.int32, (rows, c), 1)
    c_iota = c_iota_i.astype(jnp.float32)
    safe = jnp.where(ns == 1.0, 1.0, ns - 1.0)
    er_gen = c_iota * mask / safe
    er_single = jnp.where(c_iota_i == 0, 0.5, 0.0)
    er = jnp.where(ns == 1.0, er_single, er_gen)
    cl = mask * (1.0 - er)
    cr = mask * er

    m_iota = jax.lax.broadcasted_iota(jnp.int32, (rows, n), 1)
    s_l = jnp.zeros((rows, n), jnp.float32)
    s_r = jnp.zeros((rows, n), jnp.float32)
    for j in range(c):
        idx_j = idx[:, j:j + 1]
        eq = (m_iota == idx_j).astype(jnp.float32)
        s_l = s_l + cl[:, j:j + 1] * eq
        s_r = s_r + cr[:, j:j + 1] * eq

    parent = parent_ref[...]
    hl_parts = []
    hr_parts = []
    for g in range(gt):
        p_g = parent[g]
        sl_g = s_l[g * n:(g + 1) * n, :]
        sr_g = s_r[g * n:(g + 1) * n, :]
        hl_parts.append(jnp.dot(sl_g, p_g, preferred_element_type=jnp.float32))
        hr_parts.append(jnp.dot(sr_g, p_g, preferred_element_type=jnp.float32))
    h_l = jnp.concatenate(hl_parts, axis=0)
    h_r = jnp.concatenate(hr_parts, axis=0)

    p_flat = parent.reshape(rows, d)
    x = (jnp.dot(p_flat, wt_ref[...], preferred_element_type=jnp.float32)
         + jnp.dot(h_l, wl_ref[...], preferred_element_type=jnp.float32)
         + jnp.dot(h_r, wr_ref[...], preferred_element_type=jnp.float32)
         + bias_ref[...] + p_flat)

    mu = jnp.mean(x, axis=1, keepdims=True)
    var = jnp.mean(x * x, axis=1, keepdims=True) - mu * mu
    y = (x - mu) * jax.lax.rsqrt(var + 1e-5) * gamma_ref[...] + beta_ref[...]
    y = jnp.maximum(y, 0.0)
    out_ref[...] = jnp.max(y.reshape(gt, n, d), axis=1)


def kernel(parent_node_embedding, children_index, batch_tree_mask, w_t, w_l,
           w_r, bias, ln_gamma, ln_beta):
    del batch_tree_mask
    b, n, d, c, gt = _B, _N, _D, _C, _GT
    idx_flat = children_index.reshape(b * n, c)
    bias2 = bias.reshape(1, d)
    gamma2 = ln_gamma.reshape(1, d)
    beta2 = ln_beta.reshape(1, d)

    grid = (b // gt,)
    return pl.pallas_call(
        _tc_kernel,
        grid=grid,
        in_specs=[
            pl.BlockSpec((gt, n, d), lambda i: (i, 0, 0)),
            pl.BlockSpec((gt * n, c), lambda i: (i, 0)),
            pl.BlockSpec((d, d), lambda i: (0, 0)),
            pl.BlockSpec((d, d), lambda i: (0, 0)),
            pl.BlockSpec((d, d), lambda i: (0, 0)),
            pl.BlockSpec((1, d), lambda i: (0, 0)),
            pl.BlockSpec((1, d), lambda i: (0, 0)),
            pl.BlockSpec((1, d), lambda i: (0, 0)),
        ],
        out_specs=pl.BlockSpec((gt, d), lambda i: (i, 0)),
        out_shape=jax.ShapeDtypeStruct((b, d), jnp.float32),
    )(parent_node_embedding, idx_flat, w_t, w_l, w_r, bias2, gamma2, beta2)


# speedup vs baseline: 20.4040x; 1.0491x over previous
import functools
import jax
import jax.numpy as jnp
from jax import lax
from jax.experimental import pallas as pl
from jax.experimental.pallas import tpu as pltpu
from jax.experimental.pallas import tpu_sc as plsc

_B, _N, _C, _D = 256, 64, 32, 128
_GT = 8
_TREES_PER_SUBCORE = 8
_L = 16


def _coef_body(idx_ref, cl_ref, cr_ref):
    rows, c = idx_ref.shape
    idx = idx_ref[...]
    mask = (idx != 0).astype(jnp.float32)
    ns = jnp.sum(mask, axis=1, keepdims=True)
    c_iota_i = jax.lax.broadcasted_iota(jnp.int32, (rows, c), 1)
    c_iota = c_iota_i.astype(jnp.float32)
    safe = jnp.where(ns == 1.0, 1.0, ns - 1.0)
    er_gen = c_iota * mask / safe
    er_single = jnp.where(c_iota_i == 0, 0.5, 0.0)
    er = jnp.where(ns == 1.0, er_single, er_gen)
    cl_ref[...] = mask * (1.0 - er)
    cr_ref[...] = mask * er


def _coefs(idx_flat):
    rows, c = idx_flat.shape
    return pl.pallas_call(
        _coef_body,
        out_shape=(
            jax.ShapeDtypeStruct((rows, c), jnp.float32),
            jax.ShapeDtypeStruct((rows, c), jnp.float32),
        ),
    )(idx_flat)


def _bcast_lane(vec, j):
    dn = lax.GatherDimensionNumbers(offset_dims=(), collapsed_slice_dims=(0,),
                                    start_index_map=(0,))
    idx = jnp.full((_L, 1), j, jnp.int32)
    return lax.gather(vec, idx, dn, slice_sizes=(1,),
                      mode=lax.GatherScatterMode.PROMISE_IN_BOUNDS)


def _sc_body(parent_hbm, idx_hbm, cl_hbm, cr_hbm, hl_hbm, hr_hbm,
             tree2_v, idx_v, cl_v, cr_v, hl_v, hr_v):
    n, c, d, l = _N, _C, _D, _L
    wid = lax.axis_index("s") * 2 + lax.axis_index("c")

    def tree_loop(t, carry):
        b = wid * _TREES_PER_SUBCORE + t
        pltpu.sync_copy(parent_hbm.at[b], tree2_v)
        pltpu.sync_copy(idx_hbm.at[b], idx_v)
        pltpu.sync_copy(cl_hbm.at[b], cl_v)
        pltpu.sync_copy(cr_hbm.at[b], cr_v)

        def node_loop(nn, carry2):
            base = nn * c
            halves = []
            for h in range(2):
                iv = idx_v[pl.ds(base + h * l, l)]
                clv = cl_v[pl.ds(base + h * l, l)]
                crv = cr_v[pl.ds(base + h * l, l)]
                halves.append((iv, clv, crv))
            accl = [jnp.zeros((l,), jnp.float32) for _ in range(d // l)]
            accr = [jnp.zeros((l,), jnp.float32) for _ in range(d // l)]
            for iv, clv, crv in halves:
                for j in range(l):
                    bidx = _bcast_lane(iv, j)
                    bcl = _bcast_lane(clv, j)
                    bcr = _bcast_lane(crv, j)
                    lane = lax.iota(jnp.int32, l)
                    for k in range(d // l):
                        g = plsc.load_gather(tree2_v, [bidx, lane + (k * l)])
                        accl[k] = accl[k] + bcl * g
                        accr[k] = accr[k] + bcr * g
            for k in range(d // l):
                hl_v[nn, pl.ds(k * l, l)] = accl[k]
                hr_v[nn, pl.ds(k * l, l)] = accr[k]
            return carry2

        lax.fori_loop(0, n, node_loop, 0)
        pltpu.sync_copy(hl_v, hl_hbm.at[b])
        pltpu.sync_copy(hr_v, hr_hbm.at[b])
        return carry

    lax.fori_loop(0, _TREES_PER_SUBCORE, tree_loop, 0)


def _sc_gather(parent3d, idx2d, cl2d, cr2d):
    b, n, d = parent3d.shape
    mesh = plsc.VectorSubcoreMesh(core_axis_name="c", subcore_axis_name="s")
    f = functools.partial(
        pl.kernel,
        out_type=(
            jax.ShapeDtypeStruct((b, n, d), jnp.float32),
            jax.ShapeDtypeStruct((b, n, d), jnp.float32),
        ),
        mesh=mesh,
        compiler_params=pltpu.CompilerParams(needs_layout_passes=False),
        scratch_types=[
            pltpu.VMEM((n, d), jnp.float32),
            pltpu.VMEM((_N * _C,), jnp.int32),
            pltpu.VMEM((_N * _C,), jnp.float32),
            pltpu.VMEM((_N * _C,), jnp.float32),
            pltpu.VMEM((n, d), jnp.float32),
            pltpu.VMEM((n, d), jnp.float32),
        ],
    )(_sc_body)
    return f(parent3d, idx2d, cl2d, cr2d)


def _dense_body(p_ref, hl_ref, hr_ref, wt_ref, wl_ref, wr_ref, bias_ref,
                gamma_ref, beta_ref, out_ref):
    gt, n, d = _GT, _N, _D
    p = p_ref[...]
    x = (jnp.dot(p, wt_ref[...], preferred_element_type=jnp.float32)
         + jnp.dot(hl_ref[...], wl_ref[...], preferred_element_type=jnp.float32)
         + jnp.dot(hr_ref[...], wr_ref[...], preferred_element_type=jnp.float32)
         + bias_ref[...] + p)
    mu = jnp.mean(x, axis=1, keepdims=True)
    var = jnp.mean(x * x, axis=1, keepdims=True) - mu * mu
    y = (x - mu) * jax.lax.rsqrt(var + 1e-5) * gamma_ref[...] + beta_ref[...]
    y = jnp.maximum(y, 0.0)
    out_ref[...] = jnp.max(y.reshape(gt, n, d), axis=1)


def _dense(p_flat, hl_flat, hr_flat, w_t, w_l, w_r, bias2, gamma2, beta2):
    b, n, d, gt = _B, _N, _D, _GT
    rows = gt * n
    grid = (b // gt,)
    row_spec = pl.BlockSpec((rows, d), lambda i: (i, 0))
    w_spec = pl.BlockSpec((d, d), lambda i: (0, 0))
    v_spec = pl.BlockSpec((1, d), lambda i: (0, 0))
    return pl.pallas_call(
        _dense_body,
        grid=grid,
        in_specs=[row_spec, row_spec, row_spec, w_spec, w_spec, w_spec,
                  v_spec, v_spec, v_spec],
        out_specs=pl.BlockSpec((gt, d), lambda i: (i, 0)),
        out_shape=jax.ShapeDtypeStruct((b, d), jnp.float32),
    )(p_flat, hl_flat, hr_flat, w_t, w_l, w_r, bias2, gamma2, beta2)


def kernel(parent_node_embedding, children_index, batch_tree_mask, w_t, w_l,
           w_r, bias, ln_gamma, ln_beta):
    del batch_tree_mask
    b, n, d, c = _B, _N, _D, _C
    idx_flat = children_index.reshape(b * n, c)
    cl, cr = _coefs(idx_flat)

    hl3d, hr3d = _sc_gather(parent_node_embedding,
                            children_index.reshape(b, n * c),
                            cl.reshape(b, n * c),
                            cr.reshape(b, n * c))

    return _dense(parent_node_embedding.reshape(b * n, d),
                  hl3d.reshape(b * n, d),
                  hr3d.reshape(b * n, d),
                  w_t, w_l, w_r,
                  bias.reshape(1, d),
                  ln_gamma.reshape(1, d),
                  ln_beta.reshape(1, d))


# speedup vs baseline: 24.5206x; 1.2018x over previous
import functools
import jax
import jax.numpy as jnp
from jax import lax
from jax.experimental import pallas as pl
from jax.experimental.pallas import tpu as pltpu
from jax.experimental.pallas import tpu_sc as plsc

_B, _N, _C, _D = 256, 64, 32, 128
_GT = 8
_TREES_PER_SUBCORE = 8
_L = 16


def _coef_body(idx_ref, cr_ref):
    rows, c = idx_ref.shape
    idx = idx_ref[...]
    mask = (idx != 0).astype(jnp.float32)
    ns = jnp.sum(mask, axis=1, keepdims=True)
    c_iota_i = jax.lax.broadcasted_iota(jnp.int32, (rows, c), 1)
    c_iota = c_iota_i.astype(jnp.float32)
    safe = jnp.where(ns == 1.0, 1.0, ns - 1.0)
    er_gen = c_iota * mask / safe
    er_single = jnp.where(c_iota_i == 0, 0.5, 0.0)
    er = jnp.where(ns == 1.0, er_single, er_gen)
    cr_ref[...] = mask * er


def _coefs(idx_flat):
    rows, c = idx_flat.shape
    return pl.pallas_call(
        _coef_body,
        out_shape=jax.ShapeDtypeStruct((rows, c), jnp.float32),
    )(idx_flat)


def _bcast_lane(vec, j):
    dn = lax.GatherDimensionNumbers(offset_dims=(), collapsed_slice_dims=(0,),
                                    start_index_map=(0,))
    idx = jnp.full((_L, 1), j, jnp.int32)
    return lax.gather(vec, idx, dn, slice_sizes=(1,),
                      mode=lax.GatherScatterMode.PROMISE_IN_BOUNDS)


def _sc_body(parent_hbm, idx_hbm, cr_hbm, hs_hbm, hr_hbm,
             tree2_v, idx_v, cr_v, hs_v, hr_v):
    n, c, d, l = _N, _C, _D, _L
    wid = lax.axis_index("s") * 2 + lax.axis_index("c")
    zero = jnp.zeros((l,), jnp.float32)

    def tree_loop(t, carry):
        b = wid * _TREES_PER_SUBCORE + t
        pltpu.sync_copy(parent_hbm.at[b], tree2_v)
        pltpu.sync_copy(idx_hbm.at[b], idx_v)
        pltpu.sync_copy(cr_hbm.at[b], cr_v)
        for k in range(d // l):
            tree2_v[0, pl.ds(k * l, l)] = zero

        def node_loop(nn, carry2):
            base = nn * c
            halves = []
            for h in range(2):
                iv = idx_v[pl.ds(base + h * l, l)]
                crv = cr_v[pl.ds(base + h * l, l)]
                halves.append((iv, crv))
            accs = [jnp.zeros((l,), jnp.float32) for _ in range(d // l)]
            accr = [jnp.zeros((l,), jnp.float32) for _ in range(d // l)]
            for iv, crv in halves:
                for j in range(l):
                    bidx = _bcast_lane(iv, j)
                    bcr = _bcast_lane(crv, j)
                    lane = lax.iota(jnp.int32, l)
                    for k in range(d // l):
                        g = plsc.load_gather(tree2_v, [bidx, lane + (k * l)])
                        accs[k] = accs[k] + g
                        accr[k] = accr[k] + bcr * g
            for k in range(d // l):
                hs_v[nn, pl.ds(k * l, l)] = accs[k]
                hr_v[nn, pl.ds(k * l, l)] = accr[k]
            return carry2

        lax.fori_loop(0, n, node_loop, 0)
        pltpu.sync_copy(hs_v, hs_hbm.at[b])
        pltpu.sync_copy(hr_v, hr_hbm.at[b])
        return carry

    lax.fori_loop(0, _TREES_PER_SUBCORE, tree_loop, 0)


def _sc_gather(parent3d, idx2d, cr2d):
    b, n, d = parent3d.shape
    mesh = plsc.VectorSubcoreMesh(core_axis_name="c", subcore_axis_name="s")
    f = functools.partial(
        pl.kernel,
        out_type=(
            jax.ShapeDtypeStruct((b, n, d), jnp.float32),
            jax.ShapeDtypeStruct((b, n, d), jnp.float32),
        ),
        mesh=mesh,
        compiler_params=pltpu.CompilerParams(needs_layout_passes=False),
        scratch_types=[
            pltpu.VMEM((n, d), jnp.float32),
            pltpu.VMEM((_N * _C,), jnp.int32),
            pltpu.VMEM((_N * _C,), jnp.float32),
            pltpu.VMEM((n, d), jnp.float32),
            pltpu.VMEM((n, d), jnp.float32),
        ],
    )(_sc_body)
    return f(parent3d, idx2d, cr2d)


def _dense_body(p_ref, hs_ref, hr_ref, wt_ref, wl_ref, wr_ref, bias_ref,
                gamma_ref, beta_ref, out_ref):
    gt, n, d = _GT, _N, _D
    p = p_ref[...]
    w_rl = wr_ref[...] - wl_ref[...]
    x = (jnp.dot(p, wt_ref[...], preferred_element_type=jnp.float32)
         + jnp.dot(hs_ref[...], wl_ref[...], preferred_element_type=jnp.float32)
         + jnp.dot(hr_ref[...], w_rl, preferred_element_type=jnp.float32)
         + bias_ref[...] + p)
    mu = jnp.mean(x, axis=1, keepdims=True)
    var = jnp.mean(x * x, axis=1, keepdims=True) - mu * mu
    y = (x - mu) * jax.lax.rsqrt(var + 1e-5) * gamma_ref[...] + beta_ref[...]
    y = jnp.maximum(y, 0.0)
    out_ref[...] = jnp.max(y.reshape(gt, n, d), axis=1)


def _dense(p_flat, hs_flat, hr_flat, w_t, w_l, w_r, bias2, gamma2, beta2):
    b, n, d, gt = _B, _N, _D, _GT
    rows = gt * n
    grid = (b // gt,)
    row_spec = pl.BlockSpec((rows, d), lambda i: (i, 0))
    w_spec = pl.BlockSpec((d, d), lambda i: (0, 0))
    v_spec = pl.BlockSpec((1, d), lambda i: (0, 0))
    return pl.pallas_call(
        _dense_body,
        grid=grid,
        in_specs=[row_spec, row_spec, row_spec, w_spec, w_spec, w_spec,
                  v_spec, v_spec, v_spec],
        out_specs=pl.BlockSpec((gt, d), lambda i: (i, 0)),
        out_shape=jax.ShapeDtypeStruct((b, d), jnp.float32),
    )(p_flat, hs_flat, hr_flat, w_t, w_l, w_r, bias2, gamma2, beta2)


def kernel(parent_node_embedding, children_index, batch_tree_mask, w_t, w_l,
           w_r, bias, ln_gamma, ln_beta):
    del batch_tree_mask
    b, n, d, c = _B, _N, _D, _C
    idx_flat = children_index.reshape(b * n, c)
    cr = _coefs(idx_flat)

    hs3d, hr3d = _sc_gather(parent_node_embedding,
                            children_index.reshape(b, n * c),
                            cr.reshape(b, n * c))

    return _dense(parent_node_embedding.reshape(b * n, d),
                  hs3d.reshape(b * n, d),
                  hr3d.reshape(b * n, d),
                  w_t, w_l, w_r,
                  bias.reshape(1, d),
                  ln_gamma.reshape(1, d),
                  ln_beta.reshape(1, d))
